# baseline (speedup 1.0000x reference)
import functools

import jax
import jax.numpy as jnp
import numpy as np
from jax import lax
from jax.experimental import pallas as pl
from jax.experimental.pallas import tpu as pltpu
from jax.experimental.pallas import tpu_sc as plsc

DIM = 128
N_RBF = 16
CUTOFF_G = 10.0
ENV_EXP = 5
N_NODES = 10000
N_EDGES = 320000
OUT_DIM = 15

NC = 2
NS = 16
LANES = 16
NTILES = NC * NS

EPT = N_EDGES // NTILES
GEOM_CH = 2000
AGG_CH = 80
NPAD = 10240
ROWS_PER_TILE = NPAD // NS


def _sc_mesh():
    return plsc.VectorSubcoreMesh(
        core_axis_name="c", subcore_axis_name="s", num_cores=NC, num_subcores=NS
    )


def _geom_body(px_hbm, py_hbm, pz_hbm, src_hbm, dst_hbm, out_hbm,
               px_v, py_v, pz_v, sidx_v, didx_v, d2_v):
    cid = lax.axis_index("c")
    sid = lax.axis_index("s")
    tid = sid * NC + cid
    pltpu.sync_copy(px_hbm, px_v)
    pltpu.sync_copy(py_hbm, py_v)
    pltpu.sync_copy(pz_hbm, pz_v)
    for ch in range(EPT // GEOM_CH):
        base = tid * EPT + ch * GEOM_CH
        pltpu.sync_copy(src_hbm.at[pl.ds(base, GEOM_CH)], sidx_v)
        pltpu.sync_copy(dst_hbm.at[pl.ds(base, GEOM_CH)], didx_v)

        def grp(g, carry):
            sv = sidx_v[pl.ds(g * LANES, LANES)]
            dv = didx_v[pl.ds(g * LANES, LANES)]
            d2 = jnp.full((LANES,), 1e-12, jnp.float32)
            for pref in (px_v, py_v, pz_v):
                pa = plsc.load_gather(pref, [dv])
                pb = plsc.load_gather(pref, [sv])
                df = pa - pb
                d2 = d2 + df * df
            d2_v[pl.ds(g * LANES, LANES)] = d2
            return carry

        lax.fori_loop(0, GEOM_CH // LANES, grp, 0)
        pltpu.sync_copy(d2_v, out_hbm.at[pl.ds(base, GEOM_CH)])


def _sc_geom(px, py, pz, src, dst):
    return pl.kernel(
        _geom_body,
        out_type=jax.ShapeDtypeStruct((N_EDGES,), jnp.float32),
        mesh=_sc_mesh(),
        compiler_params=pltpu.CompilerParams(needs_layout_passes=False),
        scratch_types=[
            pltpu.VMEM((N_NODES,), jnp.float32),
            pltpu.VMEM((N_NODES,), jnp.float32),
            pltpu.VMEM((N_NODES,), jnp.float32),
            pltpu.VMEM((GEOM_CH,), jnp.int32),
            pltpu.VMEM((GEOM_CH,), jnp.int32),
            pltpu.VMEM((GEOM_CH,), jnp.float32),
        ],
    )(px, py, pz, src, dst)


NIDX = 6
NDAT = 3
EWW = DIM // 2


def _aggr_body(x_hbm, ew_hbm, src_hbm, dst_hbm, out_hbm, *refs):
    sidx = refs[0:NIDX]
    didx = refs[NIDX:2 * NIDX]
    xb = refs[2 * NIDX:2 * NIDX + NDAT]
    wb = refs[2 * NIDX + NDAT:2 * NIDX + 2 * NDAT]
    acc_sh = refs[2 * NIDX + 2 * NDAT]
    sems = refs[2 * NIDX + 2 * NDAT + 1:]
    si = sems[0:NIDX]
    di = sems[NIDX:2 * NIDX]
    gsem = sems[2 * NIDX:2 * NIDX + NDAT]
    wsem = sems[2 * NIDX + NDAT:2 * NIDX + 2 * NDAT]
    ssem = sems[2 * NIDX + 2 * NDAT:2 * NIDX + 3 * NDAT]

    cid = lax.axis_index("c")
    sid = lax.axis_index("s")
    tid = sid * NC + cid
    ebase = tid * EPT
    nch = EPT // AGG_CH

    def idx_start(c, k):
        cc = jnp.minimum(c, nch - 1)
        pltpu.async_copy(
            src_hbm.at[pl.ds(ebase + cc * AGG_CH, AGG_CH)], sidx[k], si[k])
        pltpu.async_copy(
            dst_hbm.at[pl.ds(ebase + cc * AGG_CH, AGG_CH)], didx[k], di[k])

    def idx_wait(k):
        pltpu.make_async_copy(
            src_hbm.at[pl.ds(ebase, AGG_CH)], sidx[k], si[k]).wait()
        pltpu.make_async_copy(
            dst_hbm.at[pl.ds(ebase, AGG_CH)], didx[k], di[k]).wait()

    def data_start(c, p, k):
        pltpu.async_copy(x_hbm.at[sidx[k]], xb[p], gsem[p])
        pltpu.async_copy(
            ew_hbm.at[pl.ds((ebase + c * AGG_CH) * EWW, AGG_CH * EWW)],
            wb[p], wsem[p])

    def data_wait(p):
        pltpu.make_async_copy(x_hbm.at[sidx[0]], xb[p], gsem[p]).wait()
        pltpu.make_async_copy(
            ew_hbm.at[pl.ds(ebase * EWW, AGG_CH * EWW)], wb[p],
            wsem[p]).wait()

    def compute(p):
        def rowfn(r, c2):
            for h in range(DIM // 32):
                wword = wb[p][pl.ds(r * EWW + h * LANES, LANES)]
                wv = plsc.bitcast(wword, jnp.bfloat16)
                wa, wc = plsc.unpack(
                    wv, format=plsc.PackFormat.INTERLEAVED,
                    preferred_element_type=jnp.float32)
                xb[p][r, pl.ds(h * 32, LANES)] = (
                    xb[p][r, pl.ds(h * 32, LANES)] * wa)
                xb[p][r, pl.ds(h * 32 + LANES, LANES)] = (
                    xb[p][r, pl.ds(h * 32 + LANES, LANES)] * wc)
            return c2

        lax.fori_loop(0, AGG_CH, rowfn, 0, unroll=4)

    def scat_start(p, k):
        pltpu.async_copy(xb[p], acc_sh.at[didx[k]], ssem[p], add=True)

    def scat_wait(p, k):
        pltpu.make_async_copy(xb[p], acc_sh.at[didx[k]], ssem[p]).wait()

    def zrow(r, c2):
        for h in range(DIM // LANES):
            xb[0][r, pl.ds(h * LANES, LANES)] = jnp.zeros(
                (LANES,), jnp.float32)
        return c2

    lax.fori_loop(0, AGG_CH, zrow, 0, unroll=8)
    for j in range(ROWS_PER_TILE // AGG_CH):
        pltpu.sync_copy(
            xb[0],
            acc_sh.at[pl.ds(sid * ROWS_PER_TILE + j * AGG_CH, AGG_CH)])

    for k in range(5):
        idx_start(k, k)
    idx_wait(0)
    data_start(0, 0, 0)
    idx_wait(1)
    data_start(1, 1, 1)
    plsc.subcore_barrier()

    def step(c, dslot, islot):
        data_wait(dslot)
        compute(dslot)
        scat_start(dslot, islot)
        if c >= 1:
            scat_wait((c - 1) % NDAT, (c - 1) % NIDX)
        idx_start(c + 5, (c + 5) % NIDX)
        if c + 2 <= nch - 1:
            idx_wait((c + 2) % NIDX)
            data_start(c + 2, (c + 2) % NDAT, (c + 2) % NIDX)

    step(0, 0, 0)
    step(1, 1, 1)

    def six(g, carry):
        c = 6 * g + 2
        for j in range(6):
            cj = c + j
            dslot = (2 + j) % NDAT
            islot = (2 + j) % NIDX
            data_wait(dslot)
            compute(dslot)
            scat_start(dslot, islot)
            scat_wait((2 + j - 1) % NDAT, (2 + j - 1) % NIDX)
            idx_start(cj + 5, (2 + j + 5) % NIDX)
            idx_wait((2 + j + 2) % NIDX)
            data_start(cj + 2, (2 + j + 2) % NDAT, (2 + j + 2) % NIDX)
        return carry

    lax.fori_loop(0, 20, six, 0)

    for c in (122, 123, 124):
        data_wait(c % NDAT)
        compute(c % NDAT)
        scat_start(c % NDAT, c % NIDX)
        scat_wait((c - 1) % NDAT, (c - 1) % NIDX)
        if c == 122:
            idx_wait(124 % NIDX)
            data_start(124, 124 % NDAT, 124 % NIDX)
    scat_wait(124 % NDAT, 124 % NIDX)
    idx_wait(125 % NIDX)
    idx_wait(126 % NIDX)

    plsc.subcore_barrier()
    pltpu.sync_copy(
        acc_sh.at[pl.ds(sid * ROWS_PER_TILE, ROWS_PER_TILE)],
        out_hbm.at[pl.ds(cid * NPAD + sid * ROWS_PER_TILE, ROWS_PER_TILE)],
    )


def _sc_aggr(x, ew, src, dst):
    return pl.kernel(
        _aggr_body,
        out_type=jax.ShapeDtypeStruct((2 * NPAD, DIM), jnp.float32),
        mesh=_sc_mesh(),
        compiler_params=pltpu.CompilerParams(needs_layout_passes=False),
        scratch_types=(
            [pltpu.VMEM((AGG_CH,), jnp.int32) for _ in range(2 * NIDX)]
            + [pltpu.VMEM((AGG_CH, DIM), jnp.float32) for _ in range(NDAT)]
            + [pltpu.VMEM((AGG_CH * EWW,), jnp.int32) for _ in range(NDAT)]
            + [pltpu.VMEM_SHARED((NPAD, DIM), jnp.float32)]
            + [pltpu.SemaphoreType.DMA for _ in range(2 * NIDX + 3 * NDAT)]
        ),
    )(x, ew, src, dst)


NB = 2000


def _init_body(posP_ref, w_ref, o_ref):
    o_ref[...] = jax.nn.relu(
        lax.dot_general(posP_ref[...], w_ref[...], (((0,), (0,)), ((), ())),
                        preferred_element_type=jnp.float32))


def _tc_init(posP, WiP):
    return pl.pallas_call(
        _init_body,
        out_shape=jax.ShapeDtypeStruct((N_NODES, DIM), jnp.float32),
    )(posP, WiP)


EB = 1024


def _edgew_body(freqs_ref, d2_ref, wrbf_ref, o_ref):
    d2 = d2_ref[...]
    d = jnp.sqrt(d2)
    dd = d * (1.0 / CUTOFF_G)
    dsafe = jnp.maximum(dd, 1e-6)
    p = ENV_EXP + 1
    ca = -(p + 1) * (p + 2) / 2.0
    cb = float(p * (p + 2))
    cc = -p * (p + 1) / 2.0
    q2 = dsafe * dsafe
    q4 = q2 * q2
    q5 = q4 * dsafe
    q6 = q5 * dsafe
    q7 = q6 * dsafe
    env = 1.0 / dsafe + ca * q5 + cb * q6 + cc * q7
    env = jnp.where(dd < 1.0, env, 0.0)
    theta = freqs_ref[0] * dd
    s1 = jnp.sin(theta)
    c2x = 2.0 * jnp.cos(theta)
    rows = [env * s1]
    sk_m1, sk = s1, c2x * s1 - 0.0
    rows.append(env * sk)
    for _ in range(2, N_RBF):
        sk_m1, sk = sk, c2x * sk - sk_m1
        rows.append(env * sk)
    s = jnp.concatenate([r.reshape(1, 8, DIM) for r in rows],
                        axis=0).reshape(N_RBF, EB)
    o_ref[...] = jax.nn.relu(
        lax.dot_general(s, wrbf_ref[...], (((0,), (0,)), ((), ())),
                        preferred_element_type=jnp.float32)
    ).astype(jnp.bfloat16)


def _tc_edgew(freqs, d2r, W_rbf):
    grid = (N_EDGES + EB - 1) // EB
    return pl.pallas_call(
        _edgew_body,
        grid=(grid,),
        in_specs=[
            pl.BlockSpec(memory_space=pltpu.SMEM),
            pl.BlockSpec((8, DIM), lambda i: (i, 0)),
            pl.BlockSpec((N_RBF, DIM), lambda i: (0, 0)),
        ],
        out_specs=pl.BlockSpec((EB, DIM), lambda i: (i, 0)),
        out_shape=jax.ShapeDtypeStruct((N_EDGES, DIM), jnp.bfloat16),
    )(freqs, d2r, W_rbf)


def _upd_body(x_ref, p_ref, wm_ref, wu_ref, o_ref):
    s = p_ref[0] + p_ref[1]
    t = jnp.dot(s, wm_ref[...], preferred_element_type=jnp.float32)
    o_ref[...] = jax.nn.relu(
        x_ref[...] + jnp.dot(t, wu_ref[...], preferred_element_type=jnp.float32))


def _tc_upd(x, p2, wm, wu):
    return pl.pallas_call(
        _upd_body,
        grid=(N_NODES // NB,),
        in_specs=[
            pl.BlockSpec((NB, DIM), lambda i: (i, 0)),
            pl.BlockSpec((2, NB, DIM), lambda i: (0, i, 0)),
            pl.BlockSpec((DIM, DIM), lambda i: (0, 0)),
            pl.BlockSpec((DIM, DIM), lambda i: (0, 0)),
        ],
        out_specs=pl.BlockSpec((NB, DIM), lambda i: (i, 0)),
        out_shape=jax.ShapeDtypeStruct((N_NODES, DIM), jnp.float32),
    )(x, p2, wm, wu)


def _updf_body(x_ref, p_ref, wm_ref, wu_ref, wo_ref, o_ref):
    s = p_ref[0] + p_ref[1]
    t = jnp.dot(s, wm_ref[...], preferred_element_type=jnp.float32)
    xn = jax.nn.relu(
        x_ref[...] + jnp.dot(t, wu_ref[...], preferred_element_type=jnp.float32))
    o_ref[...] = jnp.dot(xn, wo_ref[...], preferred_element_type=jnp.float32)


def _tc_updf(x, p2, wm, wu, wo):
    return pl.pallas_call(
        _updf_body,
        grid=(N_NODES // NB,),
        in_specs=[
            pl.BlockSpec((NB, DIM), lambda i: (i, 0)),
            pl.BlockSpec((2, NB, DIM), lambda i: (0, i, 0)),
            pl.BlockSpec((DIM, DIM), lambda i: (0, 0)),
            pl.BlockSpec((DIM, DIM), lambda i: (0, 0)),
            pl.BlockSpec((DIM, DIM), lambda i: (0, 0)),
        ],
        out_specs=pl.BlockSpec((NB, DIM), lambda i: (i, 0)),
        out_shape=jax.ShapeDtypeStruct((N_NODES, DIM), jnp.float32),
    )(x, p2, wm, wu, wo)


def kernel(pos, edge_index, W_init, freqs, W_rbf, W_msg, W_upd, W_out):
    pos = pos.astype(jnp.float32)
    src = edge_index[0]
    dst = edge_index[1]
    posT = jnp.transpose(pos)
    d2 = _sc_geom(posT[0], posT[1], posT[2], src, dst)

    posP = jnp.concatenate([posT, jnp.zeros((5, N_NODES), jnp.float32)], axis=0)
    WiP = jnp.concatenate([W_init, jnp.zeros((5, DIM), jnp.float32)], axis=0)
    x = _tc_init(posP, WiP)

    perm = np.empty((DIM,), np.int32)
    for h in range(DIM // 32):
        for j in range(16):
            perm[32 * h + 2 * j] = 32 * h + j
            perm[32 * h + 2 * j + 1] = 32 * h + 16 + j
    W_rbf_p = W_rbf[:, jnp.asarray(perm)]
    ew = _tc_edgew(freqs, d2.reshape(N_EDGES // DIM, DIM), W_rbf_p)
    ew32 = lax.bitcast_convert_type(
        ew.reshape(N_EDGES, EWW, 2), jnp.int32).reshape(-1)

    p = _sc_aggr(x, ew32, src, dst).reshape(2, NPAD, DIM)[:, :N_NODES]
    x = _tc_upd(x, p, W_msg[0], W_upd[0])
    p = _sc_aggr(x, ew32, src, dst).reshape(2, NPAD, DIM)[:, :N_NODES]
    WoP = jnp.concatenate(
        [W_out, jnp.zeros((DIM, DIM - OUT_DIM), jnp.float32)], axis=1)
    out = _tc_updf(x, p, W_msg[1], W_upd[1], WoP)
    return out[:, :OUT_DIM]


# speedup vs baseline: 2.2294x; 2.2294x over previous
import functools

import jax
import jax.numpy as jnp
import numpy as np
from jax import lax
from jax.experimental import pallas as pl
from jax.experimental.pallas import tpu as pltpu
from jax.experimental.pallas import tpu_sc as plsc

DIM = 128
N_RBF = 16
CUTOFF_G = 10.0
ENV_EXP = 5
N_NODES = 10000
N_EDGES = 320000
OUT_DIM = 15

NC = 2
NS = 16
LANES = 16
NTILES = NC * NS

EPT = N_EDGES // NTILES
GEOM_CH = 2000
AGG_CH = 80
NPAD = 10240
ROWS_PER_TILE = NPAD // NS


def _sc_mesh():
    return plsc.VectorSubcoreMesh(
        core_axis_name="c", subcore_axis_name="s", num_cores=NC, num_subcores=NS
    )


def _geom_body(px_hbm, py_hbm, pz_hbm, src_hbm, dst_hbm, out_hbm,
               px_v, py_v, pz_v, sidx_v, didx_v, d2_v):
    cid = lax.axis_index("c")
    sid = lax.axis_index("s")
    tid = sid * NC + cid
    pltpu.sync_copy(px_hbm, px_v)
    pltpu.sync_copy(py_hbm, py_v)
    pltpu.sync_copy(pz_hbm, pz_v)
    for ch in range(EPT // GEOM_CH):
        base = tid * EPT + ch * GEOM_CH
        pltpu.sync_copy(src_hbm.at[pl.ds(base, GEOM_CH)], sidx_v)
        pltpu.sync_copy(dst_hbm.at[pl.ds(base, GEOM_CH)], didx_v)

        def grp(g, carry):
            sv = sidx_v[pl.ds(g * LANES, LANES)]
            dv = didx_v[pl.ds(g * LANES, LANES)]
            d2 = jnp.full((LANES,), 1e-12, jnp.float32)
            for pref in (px_v, py_v, pz_v):
                pa = plsc.load_gather(pref, [dv])
                pb = plsc.load_gather(pref, [sv])
                df = pa - pb
                d2 = d2 + df * df
            d2_v[pl.ds(g * LANES, LANES)] = d2
            return carry

        lax.fori_loop(0, GEOM_CH // LANES, grp, 0)
        pltpu.sync_copy(d2_v, out_hbm.at[pl.ds(base, GEOM_CH)])


def _sc_geom(px, py, pz, src, dst):
    return pl.kernel(
        _geom_body,
        out_type=jax.ShapeDtypeStruct((N_EDGES,), jnp.float32),
        mesh=_sc_mesh(),
        compiler_params=pltpu.CompilerParams(needs_layout_passes=False),
        scratch_types=[
            pltpu.VMEM((N_NODES,), jnp.float32),
            pltpu.VMEM((N_NODES,), jnp.float32),
            pltpu.VMEM((N_NODES,), jnp.float32),
            pltpu.VMEM((GEOM_CH,), jnp.int32),
            pltpu.VMEM((GEOM_CH,), jnp.int32),
            pltpu.VMEM((GEOM_CH,), jnp.float32),
        ],
    )(px, py, pz, src, dst)


NIDX = 6
NDAT = 3
EWW = DIM // 2


def _aggr_body(x_hbm, ew_hbm, src_hbm, dst_hbm, out_hbm, *refs):
    sidx = refs[0:NIDX]
    didx = refs[NIDX:2 * NIDX]
    xb = refs[2 * NIDX:2 * NIDX + NDAT]
    wb = refs[2 * NIDX + NDAT:2 * NIDX + 2 * NDAT]
    acc_sh = refs[2 * NIDX + 2 * NDAT]
    sems = refs[2 * NIDX + 2 * NDAT + 1:]
    si = sems[0:NIDX]
    di = sems[NIDX:2 * NIDX]
    gsem = sems[2 * NIDX:2 * NIDX + NDAT]
    wsem = sems[2 * NIDX + NDAT:2 * NIDX + 2 * NDAT]
    ssem = sems[2 * NIDX + 2 * NDAT:2 * NIDX + 3 * NDAT]

    cid = lax.axis_index("c")
    sid = lax.axis_index("s")
    tid = sid * NC + cid
    ebase = tid * EPT
    nch = EPT // AGG_CH

    def idx_start(c, k):
        cc = jnp.minimum(c, nch - 1)
        pltpu.async_copy(
            src_hbm.at[pl.ds(ebase + cc * AGG_CH, AGG_CH)], sidx[k], si[k])
        pltpu.async_copy(
            dst_hbm.at[pl.ds(ebase + cc * AGG_CH, AGG_CH)], didx[k], di[k])

    def idx_wait(k):
        pltpu.make_async_copy(
            src_hbm.at[pl.ds(ebase, AGG_CH)], sidx[k], si[k]).wait()
        pltpu.make_async_copy(
            dst_hbm.at[pl.ds(ebase, AGG_CH)], didx[k], di[k]).wait()

    def data_start(c, p, k):
        pltpu.async_copy(x_hbm.at[sidx[k]], xb[p], gsem[p])
        pltpu.async_copy(
            ew_hbm.at[pl.ds(pl.multiple_of((ebase + c * AGG_CH) // 2, 8),
                            AGG_CH // 2)],
            wb[p], wsem[p])

    def data_wait(p):
        pltpu.make_async_copy(x_hbm.at[sidx[0]], xb[p], gsem[p]).wait()
        pltpu.make_async_copy(
            ew_hbm.at[pl.ds(pl.multiple_of(ebase // 2, 8), AGG_CH // 2)],
            wb[p], wsem[p]).wait()

    def compute(p):
        def rowfn(q, c2):
            r0 = 2 * q
            for j in range(DIM // LANES):
                wword = wb[p][q, pl.ds(j * LANES, LANES)]
                wv = plsc.bitcast(wword, jnp.bfloat16)
                wa, wc = plsc.unpack(
                    wv, format=plsc.PackFormat.INTERLEAVED,
                    preferred_element_type=jnp.float32)
                xb[p][r0, pl.ds(j * LANES, LANES)] = (
                    xb[p][r0, pl.ds(j * LANES, LANES)] * wa)
                xb[p][r0 + 1, pl.ds(j * LANES, LANES)] = (
                    xb[p][r0 + 1, pl.ds(j * LANES, LANES)] * wc)
            return c2

        lax.fori_loop(0, AGG_CH // 2, rowfn, 0, unroll=2)

    def scat_start(p, k):
        pltpu.async_copy(xb[p], acc_sh.at[didx[k]], ssem[p], add=True)

    def scat_wait(p, k):
        pltpu.make_async_copy(xb[p], acc_sh.at[didx[k]], ssem[p]).wait()

    def zrow(r, c2):
        for h in range(DIM // LANES):
            xb[0][r, pl.ds(h * LANES, LANES)] = jnp.zeros(
                (LANES,), jnp.float32)
        return c2

    lax.fori_loop(0, AGG_CH, zrow, 0, unroll=8)
    for j in range(ROWS_PER_TILE // AGG_CH):
        pltpu.sync_copy(
            xb[0],
            acc_sh.at[pl.ds(sid * ROWS_PER_TILE + j * AGG_CH, AGG_CH)])

    for k in range(5):
        idx_start(k, k)
    idx_wait(0)
    data_start(0, 0, 0)
    idx_wait(1)
    data_start(1, 1, 1)
    plsc.subcore_barrier()

    def step(c, dslot, islot):
        data_wait(dslot)
        compute(dslot)
        scat_start(dslot, islot)
        if c >= 1:
            scat_wait((c - 1) % NDAT, (c - 1) % NIDX)
        idx_start(c + 5, (c + 5) % NIDX)
        if c + 2 <= nch - 1:
            idx_wait((c + 2) % NIDX)
            data_start(c + 2, (c + 2) % NDAT, (c + 2) % NIDX)

    step(0, 0, 0)
    step(1, 1, 1)

    def six(g, carry):
        c = 6 * g + 2
        for j in range(6):
            cj = c + j
            dslot = (2 + j) % NDAT
            islot = (2 + j) % NIDX
            data_wait(dslot)
            compute(dslot)
            scat_start(dslot, islot)
            scat_wait((2 + j - 1) % NDAT, (2 + j - 1) % NIDX)
            idx_start(cj + 5, (2 + j + 5) % NIDX)
            idx_wait((2 + j + 2) % NIDX)
            data_start(cj + 2, (2 + j + 2) % NDAT, (2 + j + 2) % NIDX)
        return carry

    lax.fori_loop(0, 20, six, 0)

    for c in (122, 123, 124):
        data_wait(c % NDAT)
        compute(c % NDAT)
        scat_start(c % NDAT, c % NIDX)
        scat_wait((c - 1) % NDAT, (c - 1) % NIDX)
        if c == 122:
            idx_wait(124 % NIDX)
            data_start(124, 124 % NDAT, 124 % NIDX)
    scat_wait(124 % NDAT, 124 % NIDX)
    idx_wait(125 % NIDX)
    idx_wait(126 % NIDX)

    plsc.subcore_barrier()
    pltpu.sync_copy(
        acc_sh.at[pl.ds(sid * ROWS_PER_TILE, ROWS_PER_TILE)],
        out_hbm.at[pl.ds(cid * NPAD + sid * ROWS_PER_TILE, ROWS_PER_TILE)],
    )


def _sc_aggr(x, ew, src, dst):
    return pl.kernel(
        _aggr_body,
        out_type=jax.ShapeDtypeStruct((2 * NPAD, DIM), jnp.float32),
        mesh=_sc_mesh(),
        compiler_params=pltpu.CompilerParams(needs_layout_passes=False),
        scratch_types=(
            [pltpu.VMEM((AGG_CH,), jnp.int32) for _ in range(2 * NIDX)]
            + [pltpu.VMEM((AGG_CH, DIM), jnp.float32) for _ in range(NDAT)]
            + [pltpu.VMEM((AGG_CH // 2, DIM), jnp.int32) for _ in range(NDAT)]
            + [pltpu.VMEM_SHARED((NPAD, DIM), jnp.float32)]
            + [pltpu.SemaphoreType.DMA for _ in range(2 * NIDX + 3 * NDAT)]
        ),
    )(x, ew, src, dst)


NB = 2000


def _init_body(posP_ref, w_ref, o_ref):
    o_ref[...] = jax.nn.relu(
        lax.dot_general(posP_ref[...], w_ref[...], (((0,), (0,)), ((), ())),
                        preferred_element_type=jnp.float32))


def _tc_init(posP, WiP):
    return pl.pallas_call(
        _init_body,
        out_shape=jax.ShapeDtypeStruct((N_NODES, DIM), jnp.float32),
    )(posP, WiP)


EB = 1024


def _edgew_body(freqs_ref, d2_ref, wrbf_ref, o_ref):
    d2 = d2_ref[...]
    d = jnp.sqrt(d2)
    dd = d * (1.0 / CUTOFF_G)
    dsafe = jnp.maximum(dd, 1e-6)
    p = ENV_EXP + 1
    ca = -(p + 1) * (p + 2) / 2.0
    cb = float(p * (p + 2))
    cc = -p * (p + 1) / 2.0
    q2 = dsafe * dsafe
    q4 = q2 * q2
    q5 = q4 * dsafe
    q6 = q5 * dsafe
    q7 = q6 * dsafe
    env = 1.0 / dsafe + ca * q5 + cb * q6 + cc * q7
    env = jnp.where(dd < 1.0, env, 0.0)
    theta = freqs_ref[0] * dd
    s1 = jnp.sin(theta)
    c2x = 2.0 * jnp.cos(theta)
    rows = [env * s1]
    sk_m1, sk = s1, c2x * s1 - 0.0
    rows.append(env * sk)
    for _ in range(2, N_RBF):
        sk_m1, sk = sk, c2x * sk - sk_m1
        rows.append(env * sk)
    s = jnp.concatenate([r.reshape(1, 8, DIM) for r in rows],
                        axis=0).reshape(N_RBF, EB)
    resbf = jax.nn.relu(
        lax.dot_general(s, wrbf_ref[...], (((0,), (0,)), ((), ())),
                        preferred_element_type=jnp.float32)
    ).astype(jnp.bfloat16)
    o_ref[...] = pltpu.bitcast(resbf, jnp.int32)


def _tc_edgew(freqs, d2r, W_rbf):
    grid = (N_EDGES + EB - 1) // EB
    return pl.pallas_call(
        _edgew_body,
        grid=(grid,),
        in_specs=[
            pl.BlockSpec(memory_space=pltpu.SMEM),
            pl.BlockSpec((8, DIM), lambda i: (i, 0)),
            pl.BlockSpec((N_RBF, DIM), lambda i: (0, 0)),
        ],
        out_specs=pl.BlockSpec((EB // 2, DIM), lambda i: (i, 0)),
        out_shape=jax.ShapeDtypeStruct((N_EDGES // 2, DIM), jnp.int32),
    )(freqs, d2r, W_rbf)


UB = 1024
UPB = NPAD // UB


def _upd_body(x_ref, pa_ref, pb_ref, wm_ref, wu_ref, o_ref):
    s = pa_ref[...] + pb_ref[...]
    t = jnp.dot(s, wm_ref[...], preferred_element_type=jnp.float32)
    o_ref[...] = jax.nn.relu(
        x_ref[...] + jnp.dot(t, wu_ref[...], preferred_element_type=jnp.float32))


def _tc_upd(x, p2, wm, wu):
    return pl.pallas_call(
        _upd_body,
        grid=((N_NODES + UB - 1) // UB,),
        in_specs=[
            pl.BlockSpec((UB, DIM), lambda i: (i, 0)),
            pl.BlockSpec((UB, DIM), lambda i: (i, 0)),
            pl.BlockSpec((UB, DIM), lambda i: (UPB + i, 0)),
            pl.BlockSpec((DIM, DIM), lambda i: (0, 0)),
            pl.BlockSpec((DIM, DIM), lambda i: (0, 0)),
        ],
        out_specs=pl.BlockSpec((UB, DIM), lambda i: (i, 0)),
        out_shape=jax.ShapeDtypeStruct((N_NODES, DIM), jnp.float32),
    )(x, p2, p2, wm, wu)


def _updf_body(x_ref, pa_ref, pb_ref, wm_ref, wu_ref, wo_ref, o_ref):
    s = pa_ref[...] + pb_ref[...]
    t = jnp.dot(s, wm_ref[...], preferred_element_type=jnp.float32)
    xn = jax.nn.relu(
        x_ref[...] + jnp.dot(t, wu_ref[...], preferred_element_type=jnp.float32))
    o_ref[...] = jnp.dot(xn, wo_ref[...], preferred_element_type=jnp.float32)


def _tc_updf(x, p2, wm, wu, wo):
    return pl.pallas_call(
        _updf_body,
        grid=((N_NODES + UB - 1) // UB,),
        in_specs=[
            pl.BlockSpec((UB, DIM), lambda i: (i, 0)),
            pl.BlockSpec((UB, DIM), lambda i: (i, 0)),
            pl.BlockSpec((UB, DIM), lambda i: (UPB + i, 0)),
            pl.BlockSpec((DIM, DIM), lambda i: (0, 0)),
            pl.BlockSpec((DIM, DIM), lambda i: (0, 0)),
            pl.BlockSpec((DIM, DIM), lambda i: (0, 0)),
        ],
        out_specs=pl.BlockSpec((UB, DIM), lambda i: (i, 0)),
        out_shape=jax.ShapeDtypeStruct((N_NODES, DIM), jnp.float32),
    )(x, p2, p2, wm, wu, wo)


def kernel(pos, edge_index, W_init, freqs, W_rbf, W_msg, W_upd, W_out):
    pos = pos.astype(jnp.float32)
    src = edge_index[0]
    dst = edge_index[1]
    posT = jnp.transpose(pos)
    d2 = _sc_geom(posT[0], posT[1], posT[2], src, dst)

    posP = jnp.concatenate([posT, jnp.zeros((5, N_NODES), jnp.float32)], axis=0)
    WiP = jnp.concatenate([W_init, jnp.zeros((5, DIM), jnp.float32)], axis=0)
    x = _tc_init(posP, WiP)

    ew = _tc_edgew(freqs, d2.reshape(N_EDGES // DIM, DIM), W_rbf)

    p = _sc_aggr(x, ew, src, dst)
    x = _tc_upd(x, p, W_msg[0], W_upd[0])
    p = _sc_aggr(x, ew, src, dst)
    WoP = jnp.concatenate(
        [W_out, jnp.zeros((DIM, DIM - OUT_DIM), jnp.float32)], axis=1)
    out = _tc_updf(x, p, W_msg[1], W_upd[1], WoP)
    return out[:, :OUT_DIM]


# speedup vs baseline: 2.5878x; 1.1607x over previous
import functools

import jax
import jax.numpy as jnp
import numpy as np
from jax import lax
from jax.experimental import pallas as pl
from jax.experimental.pallas import tpu as pltpu
from jax.experimental.pallas import tpu_sc as plsc

DIM = 128
N_RBF = 16
CUTOFF_G = 10.0
ENV_EXP = 5
N_NODES = 10000
N_EDGES = 320000
OUT_DIM = 15

NC = 2
NS = 16
LANES = 16
NTILES = NC * NS

EPT = N_EDGES // NTILES
GEOM_CH = 2000
AGG_CH = 80
NPAD = 10240
ROWS_PER_TILE = NPAD // NS


def _sc_mesh():
    return plsc.VectorSubcoreMesh(
        core_axis_name="c", subcore_axis_name="s", num_cores=NC, num_subcores=NS
    )


def _geom_body(px_hbm, py_hbm, pz_hbm, src_hbm, dst_hbm, out_hbm,
               px_v, py_v, pz_v, sidx_v, didx_v, d2_v):
    cid = lax.axis_index("c")
    sid = lax.axis_index("s")
    tid = sid * NC + cid
    pltpu.sync_copy(px_hbm, px_v)
    pltpu.sync_copy(py_hbm, py_v)
    pltpu.sync_copy(pz_hbm, pz_v)
    for ch in range(EPT // GEOM_CH):
        base = tid * EPT + ch * GEOM_CH
        pltpu.sync_copy(src_hbm.at[pl.ds(base, GEOM_CH)], sidx_v)
        pltpu.sync_copy(dst_hbm.at[pl.ds(base, GEOM_CH)], didx_v)

        def grp(g, carry):
            sv = sidx_v[pl.ds(g * LANES, LANES)]
            dv = didx_v[pl.ds(g * LANES, LANES)]
            d2 = jnp.full((LANES,), 1e-12, jnp.float32)
            for pref in (px_v, py_v, pz_v):
                pa = plsc.load_gather(pref, [dv])
                pb = plsc.load_gather(pref, [sv])
                df = pa - pb
                d2 = d2 + df * df
            d2_v[pl.ds(g * LANES, LANES)] = d2
            return carry

        lax.fori_loop(0, GEOM_CH // LANES, grp, 0)
        pltpu.sync_copy(d2_v, out_hbm.at[pl.ds(base, GEOM_CH)])


def _sc_geom(px, py, pz, src, dst):
    return pl.kernel(
        _geom_body,
        out_type=jax.ShapeDtypeStruct((N_EDGES,), jnp.float32),
        mesh=_sc_mesh(),
        compiler_params=pltpu.CompilerParams(needs_layout_passes=False),
        scratch_types=[
            pltpu.VMEM((N_NODES,), jnp.float32),
            pltpu.VMEM((N_NODES,), jnp.float32),
            pltpu.VMEM((N_NODES,), jnp.float32),
            pltpu.VMEM((GEOM_CH,), jnp.int32),
            pltpu.VMEM((GEOM_CH,), jnp.int32),
            pltpu.VMEM((GEOM_CH,), jnp.float32),
        ],
    )(px, py, pz, src, dst)


NIDX = 6
NDAT = 3
EWW = DIM // 2


def _aggr_body(x_hbm, ew_hbm, src_hbm, dst_hbm, out_hbm, *refs):
    sidx = refs[0:NIDX]
    didx = refs[NIDX:2 * NIDX]
    xb = refs[2 * NIDX:2 * NIDX + NDAT]
    wb = refs[2 * NIDX + NDAT:2 * NIDX + 2 * NDAT]
    acc_sh = refs[2 * NIDX + 2 * NDAT]
    sems = refs[2 * NIDX + 2 * NDAT + 1:]
    si = sems[0:NIDX]
    di = sems[NIDX:2 * NIDX]
    gsem = sems[2 * NIDX:2 * NIDX + NDAT]
    wsem = sems[2 * NIDX + NDAT:2 * NIDX + 2 * NDAT]
    ssem = sems[2 * NIDX + 2 * NDAT:2 * NIDX + 3 * NDAT]

    cid = lax.axis_index("c")
    sid = lax.axis_index("s")
    tid = sid * NC + cid
    ebase = tid * EPT
    nch = EPT // AGG_CH

    def idx_start(c, k):
        cc = jnp.minimum(c, nch - 1)
        pltpu.async_copy(
            src_hbm.at[pl.ds(ebase + cc * AGG_CH, AGG_CH)], sidx[k], si[k])
        pltpu.async_copy(
            dst_hbm.at[pl.ds(ebase + cc * AGG_CH, AGG_CH)], didx[k], di[k])

    def idx_wait(k):
        pltpu.make_async_copy(
            src_hbm.at[pl.ds(ebase, AGG_CH)], sidx[k], si[k]).wait()
        pltpu.make_async_copy(
            dst_hbm.at[pl.ds(ebase, AGG_CH)], didx[k], di[k]).wait()

    def data_start(c, p, k):
        pltpu.async_copy(x_hbm.at[sidx[k]], xb[p], gsem[p])
        pltpu.async_copy(
            ew_hbm.at[pl.ds(pl.multiple_of((ebase + c * AGG_CH) // 2, 8),
                            AGG_CH // 2)],
            wb[p], wsem[p])

    def data_wait(p):
        pltpu.make_async_copy(x_hbm.at[sidx[0]], xb[p], gsem[p]).wait()
        pltpu.make_async_copy(
            ew_hbm.at[pl.ds(pl.multiple_of(ebase // 2, 8), AGG_CH // 2)],
            wb[p], wsem[p]).wait()

    def compute(p):
        def rowfn(q, c2):
            r0 = 2 * q
            for j in range(DIM // LANES):
                wword = wb[p][q, pl.ds(j * LANES, LANES)]
                wv = plsc.bitcast(wword, jnp.bfloat16)
                wa, wc = plsc.unpack(
                    wv, format=plsc.PackFormat.INTERLEAVED,
                    preferred_element_type=jnp.float32)
                xb[p][r0, pl.ds(j * LANES, LANES)] = (
                    xb[p][r0, pl.ds(j * LANES, LANES)] * wa)
                xb[p][r0 + 1, pl.ds(j * LANES, LANES)] = (
                    xb[p][r0 + 1, pl.ds(j * LANES, LANES)] * wc)
            return c2

        lax.fori_loop(0, AGG_CH // 2, rowfn, 0, unroll=4)

    def scat_start(p, k):
        pltpu.async_copy(xb[p], acc_sh.at[didx[k]], ssem[p], add=True)

    def scat_wait(p, k):
        pltpu.make_async_copy(xb[p], acc_sh.at[didx[k]], ssem[p]).wait()

    def zrow(r, c2):
        for h in range(DIM // LANES):
            xb[0][r, pl.ds(h * LANES, LANES)] = jnp.zeros(
                (LANES,), jnp.float32)
        return c2

    lax.fori_loop(0, AGG_CH, zrow, 0, unroll=8)
    for j in range(ROWS_PER_TILE // AGG_CH):
        pltpu.sync_copy(
            xb[0],
            acc_sh.at[pl.ds(sid * ROWS_PER_TILE + j * AGG_CH, AGG_CH)])

    for k in range(5):
        idx_start(k, k)
    idx_wait(0)
    data_start(0, 0, 0)
    idx_wait(1)
    data_start(1, 1, 1)
    plsc.subcore_barrier()

    def step(c, dslot, islot):
        data_wait(dslot)
        compute(dslot)
        scat_start(dslot, islot)
        if c >= 1:
            scat_wait((c - 1) % NDAT, (c - 1) % NIDX)
        idx_start(c + 5, (c + 5) % NIDX)
        if c + 2 <= nch - 1:
            idx_wait((c + 2) % NIDX)
            data_start(c + 2, (c + 2) % NDAT, (c + 2) % NIDX)

    step(0, 0, 0)
    step(1, 1, 1)

    def six(g, carry):
        c = 6 * g + 2
        for j in range(6):
            cj = c + j
            dslot = (2 + j) % NDAT
            islot = (2 + j) % NIDX
            data_wait(dslot)
            compute(dslot)
            scat_start(dslot, islot)
            scat_wait((2 + j - 1) % NDAT, (2 + j - 1) % NIDX)
            idx_start(cj + 5, (2 + j + 5) % NIDX)
            idx_wait((2 + j + 2) % NIDX)
            data_start(cj + 2, (2 + j + 2) % NDAT, (2 + j + 2) % NIDX)
        return carry

    lax.fori_loop(0, 20, six, 0)

    for c in (122, 123, 124):
        data_wait(c % NDAT)
        compute(c % NDAT)
        scat_start(c % NDAT, c % NIDX)
        scat_wait((c - 1) % NDAT, (c - 1) % NIDX)
        if c == 122:
            idx_wait(124 % NIDX)
            data_start(124, 124 % NDAT, 124 % NIDX)
    scat_wait(124 % NDAT, 124 % NIDX)
    idx_wait(125 % NIDX)
    idx_wait(126 % NIDX)

    plsc.subcore_barrier()
    pltpu.sync_copy(
        acc_sh.at[pl.ds(sid * ROWS_PER_TILE, ROWS_PER_TILE)],
        out_hbm.at[pl.ds(cid * NPAD + sid * ROWS_PER_TILE, ROWS_PER_TILE)],
    )


def _sc_aggr(x, ew, src, dst):
    return pl.kernel(
        _aggr_body,
        out_type=jax.ShapeDtypeStruct((2 * NPAD, DIM), jnp.float32),
        mesh=_sc_mesh(),
        compiler_params=pltpu.CompilerParams(needs_layout_passes=False),
        scratch_types=(
            [pltpu.VMEM((AGG_CH,), jnp.int32) for _ in range(2 * NIDX)]
            + [pltpu.VMEM((AGG_CH, DIM), jnp.float32) for _ in range(NDAT)]
            + [pltpu.VMEM((AGG_CH // 2, DIM), jnp.int32) for _ in range(NDAT)]
            + [pltpu.VMEM_SHARED((NPAD, DIM), jnp.float32)]
            + [pltpu.SemaphoreType.DMA for _ in range(2 * NIDX + 3 * NDAT)]
        ),
    )(x, ew, src, dst)


NB = 2000


def _init_body(posP_ref, w_ref, o_ref):
    o_ref[...] = jax.nn.relu(
        lax.dot_general(posP_ref[...], w_ref[...], (((0,), (0,)), ((), ())),
                        preferred_element_type=jnp.float32))


def _tc_init(posP, WiP):
    return pl.pallas_call(
        _init_body,
        out_shape=jax.ShapeDtypeStruct((N_NODES, DIM), jnp.float32),
    )(posP, WiP)


EB = 4096


def _edgew_body(freqs_ref, d2_ref, wrbf_ref, o_ref):
    d2 = d2_ref[...]
    d = jnp.sqrt(d2)
    dd = d * (1.0 / CUTOFF_G)
    dsafe = jnp.maximum(dd, 1e-6)
    p = ENV_EXP + 1
    ca = -(p + 1) * (p + 2) / 2.0
    cb = float(p * (p + 2))
    cc = -p * (p + 1) / 2.0
    q2 = dsafe * dsafe
    q4 = q2 * q2
    q5 = q4 * dsafe
    q6 = q5 * dsafe
    q7 = q6 * dsafe
    env = 1.0 / dsafe + ca * q5 + cb * q6 + cc * q7
    env = jnp.where(dd < 1.0, env, 0.0)
    theta = freqs_ref[0] * dd
    s1 = jnp.sin(theta)
    c2x = 2.0 * jnp.cos(theta)
    rows = [env * s1]
    sk_m1, sk = s1, c2x * s1 - 0.0
    rows.append(env * sk)
    for _ in range(2, N_RBF):
        sk_m1, sk = sk, c2x * sk - sk_m1
        rows.append(env * sk)
    s = jnp.concatenate([r.reshape(1, EB // DIM, DIM) for r in rows],
                        axis=0).reshape(N_RBF, EB)
    resbf = jax.nn.relu(
        lax.dot_general(s, wrbf_ref[...], (((0,), (0,)), ((), ())),
                        preferred_element_type=jnp.float32)
    ).astype(jnp.bfloat16)
    o_ref[...] = pltpu.bitcast(resbf, jnp.int32)


def _tc_edgew(freqs, d2r, W_rbf):
    grid = (N_EDGES + EB - 1) // EB
    return pl.pallas_call(
        _edgew_body,
        grid=(grid,),
        in_specs=[
            pl.BlockSpec(memory_space=pltpu.SMEM),
            pl.BlockSpec((EB // DIM, DIM), lambda i: (i, 0)),
            pl.BlockSpec((N_RBF, DIM), lambda i: (0, 0)),
        ],
        out_specs=pl.BlockSpec((EB // 2, DIM), lambda i: (i, 0)),
        out_shape=jax.ShapeDtypeStruct((N_EDGES // 2, DIM), jnp.int32),
    )(freqs, d2r, W_rbf)


UB = 1024
UPB = NPAD // UB


def _upd_body(x_ref, pa_ref, pb_ref, wm_ref, wu_ref, o_ref):
    s = pa_ref[...] + pb_ref[...]
    t = jnp.dot(s, wm_ref[...], preferred_element_type=jnp.float32)
    o_ref[...] = jax.nn.relu(
        x_ref[...] + jnp.dot(t, wu_ref[...], preferred_element_type=jnp.float32))


def _tc_upd(x, p2, wm, wu):
    return pl.pallas_call(
        _upd_body,
        grid=((N_NODES + UB - 1) // UB,),
        in_specs=[
            pl.BlockSpec((UB, DIM), lambda i: (i, 0)),
            pl.BlockSpec((UB, DIM), lambda i: (i, 0)),
            pl.BlockSpec((UB, DIM), lambda i: (UPB + i, 0)),
            pl.BlockSpec((DIM, DIM), lambda i: (0, 0)),
            pl.BlockSpec((DIM, DIM), lambda i: (0, 0)),
        ],
        out_specs=pl.BlockSpec((UB, DIM), lambda i: (i, 0)),
        out_shape=jax.ShapeDtypeStruct((N_NODES, DIM), jnp.float32),
    )(x, p2, p2, wm, wu)


def _updf_body(x_ref, pa_ref, pb_ref, wm_ref, wu_ref, wo_ref, o_ref):
    s = pa_ref[...] + pb_ref[...]
    t = jnp.dot(s, wm_ref[...], preferred_element_type=jnp.float32)
    xn = jax.nn.relu(
        x_ref[...] + jnp.dot(t, wu_ref[...], preferred_element_type=jnp.float32))
    o_ref[...] = jnp.dot(xn, wo_ref[...], preferred_element_type=jnp.float32)


def _tc_updf(x, p2, wm, wu, wo):
    return pl.pallas_call(
        _updf_body,
        grid=((N_NODES + UB - 1) // UB,),
        in_specs=[
            pl.BlockSpec((UB, DIM), lambda i: (i, 0)),
            pl.BlockSpec((UB, DIM), lambda i: (i, 0)),
            pl.BlockSpec((UB, DIM), lambda i: (UPB + i, 0)),
            pl.BlockSpec((DIM, DIM), lambda i: (0, 0)),
            pl.BlockSpec((DIM, DIM), lambda i: (0, 0)),
            pl.BlockSpec((DIM, DIM), lambda i: (0, 0)),
        ],
        out_specs=pl.BlockSpec((UB, DIM), lambda i: (i, 0)),
        out_shape=jax.ShapeDtypeStruct((N_NODES, DIM), jnp.float32),
    )(x, p2, p2, wm, wu, wo)


def kernel(pos, edge_index, W_init, freqs, W_rbf, W_msg, W_upd, W_out):
    pos = pos.astype(jnp.float32)
    src = edge_index[0]
    dst = edge_index[1]
    posT = jnp.transpose(pos)
    d2 = _sc_geom(posT[0], posT[1], posT[2], src, dst)

    posP = jnp.concatenate([posT, jnp.zeros((5, N_NODES), jnp.float32)], axis=0)
    WiP = jnp.concatenate([W_init, jnp.zeros((5, DIM), jnp.float32)], axis=0)
    x = _tc_init(posP, WiP)

    ew = _tc_edgew(freqs, d2.reshape(N_EDGES // DIM, DIM), W_rbf)

    p = _sc_aggr(x, ew, src, dst)
    x = _tc_upd(x, p, W_msg[0], W_upd[0])
    p = _sc_aggr(x, ew, src, dst)
    WoP = jnp.concatenate(
        [W_out, jnp.zeros((DIM, DIM - OUT_DIM), jnp.float32)], axis=1)
    out = _tc_updf(x, p, W_msg[1], W_upd[1], WoP)
    return out[:, :OUT_DIM]
